# baseline (speedup 1.0000x reference)
import functools

import jax
import jax.numpy as jnp
from jax import lax
from jax.experimental import pallas as pl
from jax.experimental.pallas import tpu as pltpu
from jax.experimental.pallas import tpu_sc as plsc

B = 16384
D = 128
L = 16
NC = 2
NS = 16
NW = NC * NS
BPW = B // NW
C = 128
NCHUNK = BPW // C


def _sigmoid(x):
    return 1.0 / (1.0 + jnp.exp(-x))


_GATHER_DNUMS = lax.GatherDimensionNumbers(
    offset_dims=(), collapsed_slice_dims=(0,), start_index_map=(0,)
)


def _shuffle(v, idx):
    return lax.gather(
        v, idx, dimension_numbers=_GATHER_DNUMS, slice_sizes=(1,),
        mode=lax.GatherScatterMode.PROMISE_IN_BOUNDS,
    )


_mesh = plsc.VectorSubcoreMesh(
    core_axis_name="c", subcore_axis_name="s", num_cores=NC, num_subcores=NS
)


@functools.partial(
    pl.kernel,
    out_type=jax.ShapeDtypeStruct((B,), jnp.float32),
    mesh=_mesh,
    scratch_types=[
        pltpu.VMEM((C,), jnp.int32),
        pltpu.VMEM((C,), jnp.int32),
        pltpu.VMEM((C, D), jnp.float32),
        pltpu.VMEM((C, D), jnp.float32),
        pltpu.VMEM((C, D), jnp.float32),
        pltpu.VMEM((C,), jnp.float32),
        pltpu.VMEM((C,), jnp.float32),
        pltpu.SemaphoreType.DMA,
        pltpu.SemaphoreType.DMA,
        pltpu.SemaphoreType.DMA,
    ],
)
def _param_sc(sid_hbm, qid_hbm, q_hbm, s_w, d_w, disc_w, out_hbm,
              sidx_v, qidx_v, s_rows, d_rows, q_rows, disc_v, out_v,
              sem_s, sem_d, sem_c):
    wid = lax.axis_index("s") * NC + lax.axis_index("c")
    lane = lax.iota(jnp.int32, L)
    perms = [jnp.reshape(lane ^ k, (L, 1)) for k in (8, 4, 2, 1)]

    def chunk_body(ch, _):
        base = wid * BPW + ch * C

        pltpu.sync_copy(sid_hbm.at[pl.ds(base, C)], sidx_v)
        pltpu.sync_copy(qid_hbm.at[pl.ds(base, C)], qidx_v)
        cp_s = pltpu.async_copy(s_w.at[sidx_v], s_rows, sem_s)
        cp_d = pltpu.async_copy(d_w.at[qidx_v], d_rows, sem_d)
        cp_c = pltpu.async_copy(disc_w.at[qidx_v], disc_v, sem_c)
        pltpu.sync_copy(q_hbm.at[pl.ds(base, C)], q_rows)
        cp_s.wait()
        cp_d.wait()
        cp_c.wait()

        def group_body(g, _):
            row_sums = jnp.zeros((L,), jnp.float32)
            for r16 in range(L):
                r = g * L + r16
                acc = jnp.zeros((L,), jnp.float32)
                for j in range(D // L):
                    s = s_rows[r, pl.ds(j * L, L)]
                    d = d_rows[r, pl.ds(j * L, L)]
                    q = q_rows[r, pl.ds(j * L, L)]
                    acc = acc + (_sigmoid(s) - _sigmoid(d)) * q
                for p in perms:
                    acc = acc + _shuffle(acc, p)
                row_sums = jnp.where(lane == r16, acc, row_sums)
            sl = pl.ds(g * L, L)
            out_v[sl] = _sigmoid(_sigmoid(disc_v[sl]) * row_sums)
            return 0

        lax.fori_loop(0, C // L, group_body, 0)

        pltpu.sync_copy(out_v, out_hbm.at[pl.ds(base, C)])
        return 0

    lax.fori_loop(0, NCHUNK, chunk_body, 0)


def kernel(student_id, question, q_matrix_line, student_emb_w, difficulty_w,
           discrimination_w):
    return _param_sc(
        student_id.astype(jnp.int32),
        question.astype(jnp.int32),
        q_matrix_line,
        student_emb_w,
        difficulty_w,
        discrimination_w.reshape(-1),
    )
